# baseline (speedup 1.0000x reference)
import functools

import jax
import jax.numpy as jnp
from jax import lax
from jax.experimental import pallas as pl
from jax.experimental.pallas import tpu as pltpu
from jax.experimental.pallas import tpu_sc as plsc

NUM_USERS = 100000
EMBED_DIM = 64
BATCH = 4096


@jax.jit
def _gather(user_idx, table):
    info = plsc.get_sparse_core_info()
    nw = info.num_cores * info.num_subcores
    b_per_w = BATCH // nw

    mesh = plsc.VectorSubcoreMesh(core_axis_name="c", subcore_axis_name="s")

    @functools.partial(
        pl.kernel,
        mesh=mesh,
        compiler_params=pltpu.CompilerParams(use_tc_tiling_on_sc=False),
        out_type=jax.ShapeDtypeStruct((BATCH, EMBED_DIM), jnp.float32),
        scratch_types=[
            pltpu.VMEM((b_per_w,), jnp.int32),
            pltpu.VMEM((b_per_w, EMBED_DIM), jnp.float32),
            pltpu.SemaphoreType.DMA,
        ],
    )
    def k(idx_hbm, table_hbm, out_hbm, idx_v, rows_v, sem):
        wid = lax.axis_index("s") * info.num_cores + lax.axis_index("c")
        base = wid * b_per_w
        pltpu.sync_copy(idx_hbm.at[pl.ds(base, b_per_w)], idx_v)
        pltpu.async_copy(table_hbm.at[idx_v], rows_v, sem).wait()
        pltpu.sync_copy(rows_v, out_hbm.at[pl.ds(base, b_per_w)])

    return k(user_idx, table)


def kernel(user_idx, table):
    return _gather(user_idx.astype(jnp.int32), table)


# speedup vs baseline: 1.4696x; 1.4696x over previous
import functools

import jax
import jax.numpy as jnp
from jax import lax
from jax.experimental import pallas as pl
from jax.experimental.pallas import tpu as pltpu
from jax.experimental.pallas import tpu_sc as plsc

NUM_USERS = 100000
EMBED_DIM = 64
BATCH = 4096


@jax.jit
def _gather(user_idx, table):
    info = plsc.get_sparse_core_info()
    nw = info.num_cores * info.num_subcores
    b_per_w = BATCH // nw

    mesh = plsc.VectorSubcoreMesh(core_axis_name="c", subcore_axis_name="s")

    @functools.partial(
        pl.kernel,
        mesh=mesh,
        out_type=jax.ShapeDtypeStruct((BATCH, EMBED_DIM), jnp.float32),
        scratch_types=[
            pltpu.VMEM((b_per_w,), jnp.int32),
            pltpu.VMEM((b_per_w, EMBED_DIM), jnp.float32),
            pltpu.SemaphoreType.DMA,
        ],
    )
    def k(idx_hbm, table_hbm, out_hbm, idx_v, rows_v, sem):
        wid = lax.axis_index("s") * info.num_cores + lax.axis_index("c")
        base = wid * b_per_w
        pltpu.sync_copy(idx_hbm.at[pl.ds(base, b_per_w)], idx_v)

        def fire(g, carry):
            vec = idx_v[pl.ds(g * 16, 16)]
            for lane in range(16):
                i = vec[lane]
                pltpu.make_async_copy(
                    table_hbm.at[pl.ds(i, 1)],
                    rows_v.at[pl.ds(g * 16 + lane, 1)],
                    sem,
                ).start()
            return carry

        lax.fori_loop(0, b_per_w // 16, fire, 0)
        pltpu.make_async_copy(
            table_hbm.at[pl.ds(0, b_per_w)], rows_v, sem
        ).wait()
        pltpu.sync_copy(rows_v, out_hbm.at[pl.ds(base, b_per_w)])

    return k(user_idx, table)


def kernel(user_idx, table):
    return _gather(user_idx.astype(jnp.int32), table)


# speedup vs baseline: 2.4805x; 1.6879x over previous
import functools

import jax
import jax.numpy as jnp
from jax import lax
from jax.experimental import pallas as pl
from jax.experimental.pallas import tpu as pltpu
from jax.experimental.pallas import tpu_sc as plsc

NUM_USERS = 100000
EMBED_DIM = 64
BATCH = 4096


@jax.jit
def _gather_t(user_idx, table_t):
    info = plsc.get_sparse_core_info()
    nw = info.num_cores * info.num_subcores
    rows_per_w = EMBED_DIM // nw

    mesh = plsc.VectorSubcoreMesh(core_axis_name="c", subcore_axis_name="s")

    @functools.partial(
        pl.kernel,
        mesh=mesh,
        compiler_params=pltpu.CompilerParams(needs_layout_passes=False),
        out_type=jax.ShapeDtypeStruct((EMBED_DIM, BATCH), jnp.float32),
        scratch_types=[
            pltpu.VMEM((BATCH,), jnp.int32),
            pltpu.VMEM((NUM_USERS,), jnp.float32),
            pltpu.VMEM((BATCH,), jnp.float32),
        ],
    )
    def k(idx_hbm, t_hbm, out_hbm, idx_v, row_v, orow_v):
        wid = lax.axis_index("s") * info.num_cores + lax.axis_index("c")
        pltpu.sync_copy(idx_hbm, idx_v)
        for p in range(rows_per_w):
            j = wid * rows_per_w + p
            pltpu.sync_copy(t_hbm.at[j], row_v)

            def gath(g, carry):
                iv = idx_v[pl.ds(g * 16, 16)]
                orow_v[pl.ds(g * 16, 16)] = plsc.load_gather(row_v, [iv])
                return carry

            lax.fori_loop(0, BATCH // 16, gath, 0)
            pltpu.sync_copy(orow_v, out_hbm.at[j])

    return k(user_idx, table_t)


def kernel(user_idx, table):
    out_t = _gather_t(user_idx.astype(jnp.int32), table.T)
    return out_t.T


# speedup vs baseline: 2.4853x; 1.0019x over previous
import functools

import jax
import jax.numpy as jnp
from jax import lax
from jax.experimental import pallas as pl
from jax.experimental.pallas import tpu as pltpu
from jax.experimental.pallas import tpu_sc as plsc

NUM_USERS = 100000
EMBED_DIM = 64
BATCH = 4096


@jax.jit
def _gather_t(user_idx, table_t):
    info = plsc.get_sparse_core_info()
    nw = info.num_cores * info.num_subcores
    rows_per_w = EMBED_DIM // nw

    mesh = plsc.VectorSubcoreMesh(core_axis_name="c", subcore_axis_name="s")

    @functools.partial(
        pl.kernel,
        mesh=mesh,
        compiler_params=pltpu.CompilerParams(
            needs_layout_passes=False, skip_device_barrier=True
        ),
        out_type=jax.ShapeDtypeStruct((EMBED_DIM, BATCH), jnp.float32),
        scratch_types=[
            pltpu.VMEM((BATCH,), jnp.int32),
            pltpu.VMEM((NUM_USERS,), jnp.float32),
            pltpu.VMEM((BATCH,), jnp.float32),
        ],
    )
    def k(idx_hbm, t_hbm, out_hbm, idx_v, row_v, orow_v):
        wid = lax.axis_index("s") * info.num_cores + lax.axis_index("c")
        pltpu.sync_copy(idx_hbm, idx_v)
        for p in range(rows_per_w):
            j = wid * rows_per_w + p
            pltpu.sync_copy(t_hbm.at[j], row_v)

            def gath(g, carry):
                iv = idx_v[pl.ds(g * 16, 16)]
                orow_v[pl.ds(g * 16, 16)] = plsc.load_gather(row_v, [iv])
                return carry

            lax.fori_loop(0, BATCH // 16, gath, 0)
            pltpu.sync_copy(orow_v, out_hbm.at[j])

    return k(user_idx, table_t)


def kernel(user_idx, table):
    out_t = _gather_t(user_idx.astype(jnp.int32), table.T)
    return out_t.T
